# Initial kernel scaffold; baseline (speedup 1.0000x reference)
import jax
import jax.numpy as jnp
from jax.experimental import pallas as pl


def kernel(x, gate_w1, gate_b1, gate_w2, gate_b2, weight_main, U_experts, S_experts, V_experts, bias):
    raise NotImplementedError("write your pallas kernel here")


# speedup vs baseline: 6.3447x; 6.3447x over previous
import functools

import jax
import jax.numpy as jnp
from jax import lax
from jax.experimental import pallas as pl
from jax.experimental.pallas import tpu as pltpu

IN_F = 1024
OUT_F = 1024
NUM_EXPERTS = 8
RANK = 64
TOP_K = 2
GATE_HID = 256
N_TOK = 8192

BLK = 512
NEG_INF = -3.0e38


def _moe_kernel(x_ref, gw1_ref, gb1_ref, gw2_ref, gb2_ref, wm_ref,
                vflat_ref, sflat_ref, ut_ref, bias_ref, emat_ref,
                out_ref, bal_ref, cnt_acc, ps_acc):
    i = pl.program_id(0)
    n_steps = pl.num_programs(0)
    x = x_ref[...]

    h = lax.dot_general(x, gw1_ref[...], (((1,), (1,)), ((), ())),
                        preferred_element_type=jnp.float32)
    h = jnp.maximum(h + gb1_ref[...], 0.0)
    logits = lax.dot_general(h, gw2_ref[...], (((1,), (1,)), ((), ())),
                             preferred_element_type=jnp.float32)
    logits = logits + gb2_ref[...]

    eid = lax.broadcasted_iota(jnp.int32, (BLK, NUM_EXPERTS), 1)
    m1 = jnp.max(logits, axis=1, keepdims=True)
    cand1 = jnp.where(logits == m1, eid, NUM_EXPERTS)
    idx1 = jnp.min(cand1, axis=1, keepdims=True)
    mask1 = (eid == idx1).astype(jnp.float32)
    masked = jnp.where(eid == idx1, NEG_INF, logits)
    m2 = jnp.max(masked, axis=1, keepdims=True)
    cand2 = jnp.where(masked == m2, eid, NUM_EXPERTS)
    idx2 = jnp.min(cand2, axis=1, keepdims=True)
    mask2 = (eid == idx2).astype(jnp.float32)

    e2 = jnp.exp(m2 - m1)
    denom = 1.0 + e2
    g1 = 1.0 / denom
    g2 = e2 / denom
    comb = g1 * mask1 + g2 * mask2

    probs = jnp.exp(logits - m1)
    probs = probs / jnp.sum(probs, axis=1, keepdims=True)
    cnt_part = jnp.sum(mask1 + mask2, axis=0, keepdims=True)
    ps_part = jnp.sum(probs, axis=0, keepdims=True)

    @pl.when(i == 0)
    def _init():
        cnt_acc[...] = cnt_part
        ps_acc[...] = ps_part

    @pl.when(i > 0)
    def _acc():
        cnt_acc[...] += cnt_part
        ps_acc[...] += ps_part

    @pl.when(i == n_steps - 1)
    def _fin():
        n = jnp.float32(n_steps * BLK)
        tpe = cnt_acc[...] / n
        rpe = ps_acc[...] / n
        bal_ref[...] = jnp.float32(NUM_EXPERTS) * jnp.sum(
            tpe * rpe, axis=1, keepdims=True)

    t = lax.dot_general(x, vflat_ref[...], (((1,), (1,)), ((), ())),
                        preferred_element_type=jnp.float32)
    comb_exp = lax.dot_general(comb, emat_ref[...], (((1,), (0,)), ((), ())),
                               preferred_element_type=jnp.float32)
    t = t * sflat_ref[...] * comb_exp
    eo = lax.dot_general(t, ut_ref[...], (((1,), (0,)), ((), ())),
                         preferred_element_type=jnp.float32)

    main = lax.dot_general(x, wm_ref[...], (((1,), (1,)), ((), ())),
                           preferred_element_type=jnp.float32)
    out_ref[...] = main + eo + bias_ref[...]


@jax.jit
def kernel(x, gate_w1, gate_b1, gate_w2, gate_b2, weight_main,
           U_experts, S_experts, V_experts, bias):
    n = x.shape[0]
    v_flat = V_experts.reshape(NUM_EXPERTS * RANK, IN_F)
    u_t = jnp.transpose(U_experts, (0, 2, 1)).reshape(NUM_EXPERTS * RANK, OUT_F)
    s_flat = S_experts.reshape(1, NUM_EXPERTS * RANK)
    emat = (jnp.arange(NUM_EXPERTS, dtype=jnp.int32)[:, None] ==
            (jnp.arange(NUM_EXPERTS * RANK, dtype=jnp.int32)[None, :] // RANK)
            ).astype(jnp.float32)

    grid = (n // BLK,)
    const = lambda i: (0, 0)
    out, bal = pl.pallas_call(
        _moe_kernel,
        grid=grid,
        in_specs=[
            pl.BlockSpec((BLK, IN_F), lambda i: (i, 0)),
            pl.BlockSpec((GATE_HID, IN_F), const),
            pl.BlockSpec((1, GATE_HID), const),
            pl.BlockSpec((NUM_EXPERTS, GATE_HID), const),
            pl.BlockSpec((1, NUM_EXPERTS), const),
            pl.BlockSpec((OUT_F, IN_F), const),
            pl.BlockSpec((NUM_EXPERTS * RANK, IN_F), const),
            pl.BlockSpec((1, NUM_EXPERTS * RANK), const),
            pl.BlockSpec((NUM_EXPERTS * RANK, OUT_F), const),
            pl.BlockSpec((1, OUT_F), const),
            pl.BlockSpec((NUM_EXPERTS, NUM_EXPERTS * RANK), const),
        ],
        out_specs=[
            pl.BlockSpec((BLK, OUT_F), lambda i: (i, 0)),
            pl.BlockSpec((1, 1), const),
        ],
        out_shape=[
            jax.ShapeDtypeStruct((n, OUT_F), jnp.float32),
            jax.ShapeDtypeStruct((1, 1), jnp.float32),
        ],
        scratch_shapes=[
            pltpu.VMEM((1, NUM_EXPERTS), jnp.float32),
            pltpu.VMEM((1, NUM_EXPERTS), jnp.float32),
        ],
    )(x, gate_w1, gate_b1.reshape(1, GATE_HID), gate_w2,
      gate_b2.reshape(1, NUM_EXPERTS), weight_main, v_flat, s_flat, u_t,
      bias.reshape(1, OUT_F), emat)
    return out, bal[0, 0]


# speedup vs baseline: 7.0857x; 1.1168x over previous
import functools

import jax
import jax.numpy as jnp
from jax import lax
from jax.experimental import pallas as pl
from jax.experimental.pallas import tpu as pltpu

IN_F = 1024
OUT_F = 1024
NUM_EXPERTS = 8
RANK = 64
TOP_K = 2
GATE_HID = 256
N_TOK = 8192

BLK = 512
NEG_INF = -3.0e38


def _moe_kernel(x_ref, gw1_ref, gb1_ref, gw2_ref, gb2_ref, wm_ref,
                vflat_ref, sflat_ref, ut_ref, bias_ref, emat_ref,
                out_ref, bal_ref, cnt_acc, ps_acc):
    i = pl.program_id(0)
    n_steps = pl.num_programs(0)
    x = x_ref[...]

    h = lax.dot_general(x, gw1_ref[...], (((1,), (1,)), ((), ())),
                        preferred_element_type=jnp.float32)
    h = jnp.maximum(h + gb1_ref[...], 0.0)
    logits = lax.dot_general(gw2_ref[...], h, (((1,), (1,)), ((), ())),
                             preferred_element_type=jnp.float32)
    logits = logits + gb2_ref[...]

    eid = lax.broadcasted_iota(jnp.int32, (NUM_EXPERTS, BLK), 0)
    m1 = jnp.max(logits, axis=0, keepdims=True)
    cand1 = jnp.where(logits == m1, eid, NUM_EXPERTS)
    idx1 = jnp.min(cand1, axis=0, keepdims=True)
    mask1 = (eid == idx1).astype(jnp.float32)
    masked = jnp.where(eid == idx1, NEG_INF, logits)
    m2 = jnp.max(masked, axis=0, keepdims=True)
    cand2 = jnp.where(masked == m2, eid, NUM_EXPERTS)
    idx2 = jnp.min(cand2, axis=0, keepdims=True)
    mask2 = (eid == idx2).astype(jnp.float32)

    e2 = jnp.exp(m2 - m1)
    denom = 1.0 + e2
    g1 = 1.0 / denom
    g2 = e2 / denom
    comb = g1 * mask1 + g2 * mask2

    probs = jnp.exp(logits - m1)
    probs = probs / jnp.sum(probs, axis=0, keepdims=True)
    cnt_part = jnp.sum(mask1 + mask2, axis=1, keepdims=True)
    ps_part = jnp.sum(probs, axis=1, keepdims=True)

    @pl.when(i == 0)
    def _init():
        cnt_acc[...] = cnt_part
        ps_acc[...] = ps_part

    @pl.when(i > 0)
    def _acc():
        cnt_acc[...] += cnt_part
        ps_acc[...] += ps_part

    @pl.when(i == n_steps - 1)
    def _fin():
        n = jnp.float32(n_steps * BLK)
        tpe = cnt_acc[...] / n
        rpe = ps_acc[...] / n
        bal_ref[...] = jnp.float32(NUM_EXPERTS) * jnp.sum(
            tpe * rpe, axis=0, keepdims=True)

    t = lax.dot_general(x, vflat_ref[...], (((1,), (1,)), ((), ())),
                        preferred_element_type=jnp.float32)
    comb_exp = lax.dot_general(comb, emat_ref[...], (((0,), (0,)), ((), ())),
                               preferred_element_type=jnp.float32)
    t = t * sflat_ref[...] * comb_exp
    eo = lax.dot_general(t, ut_ref[...], (((1,), (0,)), ((), ())),
                         preferred_element_type=jnp.float32)

    main = lax.dot_general(x, wm_ref[...], (((1,), (1,)), ((), ())),
                           preferred_element_type=jnp.float32)
    out_ref[...] = main + eo + bias_ref[...]


@jax.jit
def kernel(x, gate_w1, gate_b1, gate_w2, gate_b2, weight_main,
           U_experts, S_experts, V_experts, bias):
    n = x.shape[0]
    v_flat = V_experts.reshape(NUM_EXPERTS * RANK, IN_F)
    u_t = jnp.transpose(U_experts, (0, 2, 1)).reshape(NUM_EXPERTS * RANK, OUT_F)
    s_flat = S_experts.reshape(1, NUM_EXPERTS * RANK)
    emat = (jnp.arange(NUM_EXPERTS, dtype=jnp.int32)[:, None] ==
            (jnp.arange(NUM_EXPERTS * RANK, dtype=jnp.int32)[None, :] // RANK)
            ).astype(jnp.float32)

    grid = (n // BLK,)
    const = lambda i: (0, 0)
    out, bal = pl.pallas_call(
        _moe_kernel,
        grid=grid,
        in_specs=[
            pl.BlockSpec((BLK, IN_F), lambda i: (i, 0)),
            pl.BlockSpec((GATE_HID, IN_F), const),
            pl.BlockSpec((1, GATE_HID), const),
            pl.BlockSpec((NUM_EXPERTS, GATE_HID), const),
            pl.BlockSpec((NUM_EXPERTS, 1), const),
            pl.BlockSpec((OUT_F, IN_F), const),
            pl.BlockSpec((NUM_EXPERTS * RANK, IN_F), const),
            pl.BlockSpec((1, NUM_EXPERTS * RANK), const),
            pl.BlockSpec((NUM_EXPERTS * RANK, OUT_F), const),
            pl.BlockSpec((1, OUT_F), const),
            pl.BlockSpec((NUM_EXPERTS, NUM_EXPERTS * RANK), const),
        ],
        out_specs=[
            pl.BlockSpec((BLK, OUT_F), lambda i: (i, 0)),
            pl.BlockSpec((1, 1), const),
        ],
        out_shape=[
            jax.ShapeDtypeStruct((n, OUT_F), jnp.float32),
            jax.ShapeDtypeStruct((1, 1), jnp.float32),
        ],
        scratch_shapes=[
            pltpu.VMEM((NUM_EXPERTS, 1), jnp.float32),
            pltpu.VMEM((NUM_EXPERTS, 1), jnp.float32),
        ],
    )(x, gate_w1, gate_b1.reshape(1, GATE_HID), gate_w2,
      gate_b2.reshape(NUM_EXPERTS, 1), weight_main, v_flat, s_flat, u_t,
      bias.reshape(1, OUT_F), emat)
    return out, bal[0, 0]
